# Initial kernel scaffold; baseline (speedup 1.0000x reference)
import jax
import jax.numpy as jnp
from jax.experimental import pallas as pl


def kernel(x, Wg, bg, We, be):
    raise NotImplementedError("write your pallas kernel here")


# speedup vs baseline: 12.3487x; 12.3487x over previous
import functools

import jax
import jax.numpy as jnp
from jax.experimental import pallas as pl
from jax.experimental.pallas import tpu as pltpu


def _moe_kernel(x_ref, wg_ref, bg_ref, we_ref, be_ref, out_ref):
    B, S, D = x_ref.shape
    E = wg_ref.shape[1]
    x3 = x_ref[...]
    x2 = x3.reshape(B * S, D)
    logits = (
        jnp.dot(x2, wg_ref[...], preferred_element_type=jnp.float32)
        + bg_ref[...]
    )

    lane = jax.lax.broadcasted_iota(jnp.int32, logits.shape, 1)
    m1 = jnp.max(logits, axis=-1, keepdims=True)
    eq1 = logits == m1
    idx1 = jnp.min(jnp.where(eq1, lane, E), axis=-1, keepdims=True)
    mask1 = lane == idx1
    masked = jnp.where(mask1, -jnp.inf, logits)
    m2 = jnp.max(masked, axis=-1, keepdims=True)
    eq2 = masked == m2
    idx2 = jnp.min(jnp.where(eq2, lane, E), axis=-1, keepdims=True)
    mask2 = lane == idx2
    sel = mask1 | mask2
    denom = 1.0 + jnp.exp(m2 - m1)
    w = jnp.where(sel, jnp.exp(logits - m1), 0.0) / denom

    w3 = w.reshape(B, S, E)
    c = jnp.sum(w3, axis=1)
    ys = []
    for b in range(B):
        yb = jax.lax.dot_general(
            w3[b], x3[b], (((0,), (0,)), ((), ())),
            preferred_element_type=jnp.float32,
        )
        ys.append(yb)
    y = jnp.stack(ys, axis=0).reshape(B, E * D)
    out = (
        jnp.dot(y, we_ref[...], preferred_element_type=jnp.float32)
        + jnp.dot(c, be_ref[...], preferred_element_type=jnp.float32)
    )
    out_ref[...] = out


@functools.partial(jax.jit, static_argnames=())
def kernel(x, Wg, bg, We, be):
    B, S, D = x.shape
    E = Wg.shape[1]
    O = We.shape[2]
    We_flat = We.reshape(E * D, O)
    bg2 = bg.reshape(1, E)
    return pl.pallas_call(
        _moe_kernel,
        out_shape=jax.ShapeDtypeStruct((B, O), jnp.float32),
        compiler_params=pltpu.CompilerParams(
            vmem_limit_bytes=110 * 1024 * 1024,
        ),
    )(x, Wg, bg2, We_flat, be)
